# Initial kernel scaffold; baseline (speedup 1.0000x reference)
import jax
import jax.numpy as jnp
from jax.experimental import pallas as pl


def kernel(x_graph, edge_index, batch, x_tabular, W_g1, b_g1, W_g2, b_g2, W_g3, b_g3, W_g4, b_g4, W_t1, b_t1, W_t2, b_t2, W_f1, b_f1, W_f2, b_f2, W_f3, b_f3):
    raise NotImplementedError("write your pallas kernel here")


# speedup vs baseline: 6.9301x; 6.9301x over previous
import functools

import jax
import jax.numpy as jnp
from jax import lax
from jax.experimental import pallas as pl
from jax.experimental.pallas import tpu as pltpu
from jax.experimental.pallas import tpu_sc as plsc

N = 10000
NPAD = 10240
F = 128
E = 320000
CHUNK = 128
NCHUNK = 79
EPAD = 32 * NCHUNK * CHUNK
G = 64
GP = 72
RPT = NPAD // 32
POOL_SUB = 64

_mesh = plsc.VectorSubcoreMesh(core_axis_name="c", subcore_axis_name="s")


def _indeg_kernel(dst_hbm, z_hbm, out_hbm, dst_v, ind_v):
    c = lax.axis_index("c")
    s = lax.axis_index("s")
    w = c * 16 + s
    pltpu.sync_copy(z_hbm, ind_v)
    pltpu.sync_copy(dst_hbm.at[w], dst_v)
    ones = jnp.ones((16,), jnp.float32)

    def body_j(j, carry):
        def body_k(k, carry2):
            idx = dst_v[j, pl.ds(k * 16, 16)]
            plsc.addupdate_scatter(ind_v, [idx], ones)
            return carry2
        return lax.fori_loop(0, CHUNK // 16, body_k, carry)

    lax.fori_loop(0, NCHUNK, body_j, 0)
    pltpu.sync_copy(ind_v, out_hbm.at[w])


def _sc_indeg(dstp, zeros1d):
    k = functools.partial(
        pl.kernel,
        out_type=jax.ShapeDtypeStruct((32, NPAD), jnp.float32),
        mesh=_mesh,
        compiler_params=pltpu.CompilerParams(needs_layout_passes=False),
        scratch_types=[
            pltpu.VMEM((NCHUNK, CHUNK), jnp.int32),
            pltpu.VMEM((NPAD,), jnp.float32),
        ],
    )(_indeg_kernel)
    return k(dstp, zeros1d)


def _msg_kernel(hp_hbm, src_hbm, dst_hbm, z_hbm, out_hbm,
                src_v, dst_v, buf, acc_sh, sem):
    c = lax.axis_index("c")
    s = lax.axis_index("s")
    w = c * 16 + s
    pltpu.sync_copy(z_hbm, acc_sh.at[pl.ds(s * RPT, RPT)])
    pltpu.sync_copy(src_hbm.at[w], src_v)
    pltpu.sync_copy(dst_hbm.at[w], dst_v)
    plsc.subcore_barrier()

    def body_j(j, carry):
        pltpu.async_copy(hp_hbm.at[src_v.at[j]], buf, sem).wait()
        pltpu.sync_copy(buf, acc_sh.at[dst_v.at[j]], add=True)
        return carry

    lax.fori_loop(0, NCHUNK, body_j, 0)
    plsc.subcore_barrier()
    pltpu.sync_copy(acc_sh.at[pl.ds(s * RPT, RPT)],
                    out_hbm.at[c].at[pl.ds(s * RPT, RPT)])


def _sc_msg(hp, srcp, dstp, zrows):
    k = functools.partial(
        pl.kernel,
        out_type=jax.ShapeDtypeStruct((2, NPAD, F), jnp.float32),
        mesh=_mesh,
        compiler_params=pltpu.CompilerParams(needs_layout_passes=False),
        scratch_types=[
            pltpu.VMEM((NCHUNK, CHUNK), jnp.int32),
            pltpu.VMEM((NCHUNK, CHUNK), jnp.int32),
            pltpu.VMEM((CHUNK, F), jnp.float32),
            pltpu.VMEM_SHARED((NPAD, F), jnp.float32),
            pltpu.SemaphoreType.DMA,
        ],
    )(_msg_kernel)
    return k(hp, srcp, dstp, zrows)


def _pool_kernel(s0_hbm, s1_hbm, hp_hbm, d_hbm, b_hbm, bt_hbm, z_hbm,
                 sum_hbm, max_hbm,
                 s0_v, s1_v, hp_v, d_v, b_v, bt_v, sacc, macc):
    c = lax.axis_index("c")
    s = lax.axis_index("s")
    w = c * 16 + s
    base = w * RPT
    pltpu.sync_copy(z_hbm, sacc)
    pltpu.sync_copy(z_hbm, macc)
    pltpu.sync_copy(d_hbm.at[pl.ds(base, RPT)], d_v)
    pltpu.sync_copy(bt_hbm.at[pl.ds(base, RPT)], bt_v)
    pltpu.sync_copy(b_hbm, b_v)

    def body_chunk(ch, carry):
        pltpu.sync_copy(s0_hbm.at[pl.ds(base + ch * POOL_SUB, POOL_SUB)], s0_v)
        pltpu.sync_copy(s1_hbm.at[pl.ds(base + ch * POOL_SUB, POOL_SUB)], s1_v)
        pltpu.sync_copy(hp_hbm.at[pl.ds(base + ch * POOL_SUB, POOL_SUB)], hp_v)

        def body_q(q, carry2):
            rr = ch * POOL_SUB + q * 16
            g16 = bt_v[pl.ds(rr, 16)]
            d16 = d_v[pl.ds(rr, 16)]
            for r2 in range(16):
                r = q * 16 + r2
                dv16 = jnp.broadcast_to(d16[r2], (16,))
                off = pl.multiple_of(g16[r2] * F, F)
                for kk in range(F // 16):
                    sl = pl.ds(kk * 16, 16)
                    v = dv16 * (s0_v[r, sl] + s1_v[r, sl] + 2.0 * hp_v[r, sl])
                    v = jnp.maximum(v + b_v[sl], 0.0)
                    osl = pl.ds(off + kk * 16, 16)
                    sacc[osl] = sacc[osl] + v
                    macc[osl] = jnp.maximum(macc[osl], v)
            return carry2

        return lax.fori_loop(0, POOL_SUB // 16, body_q, carry)

    lax.fori_loop(0, RPT // POOL_SUB, body_chunk, 0)
    pltpu.sync_copy(sacc, sum_hbm.at[w])
    pltpu.sync_copy(macc, max_hbm.at[w])


def _sc_pool(s0, s1, hp, dcol, b4, batchp, zrows):
    k = functools.partial(
        pl.kernel,
        out_type=(
            jax.ShapeDtypeStruct((32, GP * F), jnp.float32),
            jax.ShapeDtypeStruct((32, GP * F), jnp.float32),
        ),
        mesh=_mesh,
        compiler_params=pltpu.CompilerParams(needs_layout_passes=False),
        scratch_types=[
            pltpu.VMEM((POOL_SUB, F), jnp.float32),
            pltpu.VMEM((POOL_SUB, F), jnp.float32),
            pltpu.VMEM((POOL_SUB, F), jnp.float32),
            pltpu.VMEM((RPT,), jnp.float32),
            pltpu.VMEM((F,), jnp.float32),
            pltpu.VMEM((RPT,), jnp.int32),
            pltpu.VMEM((GP * F,), jnp.float32),
            pltpu.VMEM((GP * F,), jnp.float32),
        ],
    )(_pool_kernel)
    zflat = zrows.reshape(-1)[: GP * F]
    return k(s0, s1, hp, dcol, b4, batchp, zflat)


def _mm1_body(indT, xg, W, hp_out, d_out):
    ind = jnp.sum(indT[...], axis=1, keepdims=True)
    dv = lax.rsqrt(ind + 2.0)
    i = pl.program_id(0)
    rows = i * 128 + lax.broadcasted_iota(jnp.int32, (128, 1), 0)
    dv = jnp.where(rows < N, dv, 0.0)
    d_out[...] = dv
    hp_out[...] = jnp.dot(xg[...] * dv, W[...],
                          preferred_element_type=jnp.float32)


def _tc_mm1(indegT, xgpad, W1):
    return pl.pallas_call(
        _mm1_body,
        grid=(NPAD // 128,),
        in_specs=[
            pl.BlockSpec((128, 32), lambda i: (i, 0)),
            pl.BlockSpec((128, F), lambda i: (i, 0)),
            pl.BlockSpec((F, F), lambda i: (0, 0)),
        ],
        out_specs=[
            pl.BlockSpec((128, F), lambda i: (i, 0)),
            pl.BlockSpec((128, 1), lambda i: (i, 0)),
        ],
        out_shape=[
            jax.ShapeDtypeStruct((NPAD, F), jnp.float32),
            jax.ShapeDtypeStruct((NPAD, 1), jnp.float32),
        ],
    )(indegT, xgpad, W1)


def _mid_body(s0, s1, hp, dv, b, W, out):
    x = jnp.maximum(dv[...] * (s0[...] + s1[...] + 2.0 * hp[...]) + b[...],
                    0.0)
    out[...] = jnp.dot(x * dv[...], W[...],
                       preferred_element_type=jnp.float32)


def _tc_mid(s0, s1, hp, dcol, brow, W):
    return pl.pallas_call(
        _mid_body,
        grid=(NPAD // 128,),
        in_specs=[
            pl.BlockSpec((128, F), lambda i: (i, 0)),
            pl.BlockSpec((128, F), lambda i: (i, 0)),
            pl.BlockSpec((128, F), lambda i: (i, 0)),
            pl.BlockSpec((128, 1), lambda i: (i, 0)),
            pl.BlockSpec((1, F), lambda i: (0, 0)),
            pl.BlockSpec((F, F), lambda i: (0, 0)),
        ],
        out_specs=pl.BlockSpec((128, F), lambda i: (i, 0)),
        out_shape=jax.ShapeDtypeStruct((NPAD, F), jnp.float32),
    )(s0, s1, hp, dcol, brow, W)


def _final_body(sumP, maxP, b2, xt, Wt1, bt1, Wt2, bt2,
                Wf1, bf1, Wf2, bf2, Wf3, bf3, out):
    sums = jnp.sum(sumP[...], axis=0)[:G]
    maxs = jnp.max(maxP[...], axis=0)[:G]
    gi = lax.broadcasted_iota(jnp.int32, (G, NPAD // 128, 128), 0)
    cnt = jnp.sum((b2[...][None, :, :] == gi).astype(jnp.float32),
                  axis=(1, 2))
    mean = sums / jnp.maximum(cnt, 1.0)[:, None]
    t = jnp.maximum(jnp.dot(xt[...], Wt1[...],
                            preferred_element_type=jnp.float32) + bt1[...],
                    0.0)
    x2 = jnp.maximum(jnp.dot(t, Wt2[...],
                             preferred_element_type=jnp.float32) + bt2[...],
                     0.0)
    Wf1v = Wf1[...]
    h1 = (jnp.dot(mean, Wf1v[0:F], preferred_element_type=jnp.float32)
          + jnp.dot(maxs, Wf1v[F:2 * F], preferred_element_type=jnp.float32)
          + jnp.dot(x2, Wf1v[2 * F:], preferred_element_type=jnp.float32)
          + bf1[...])
    h1 = jnp.maximum(h1, 0.0)
    h2 = jnp.maximum(jnp.dot(h1, Wf2[...],
                             preferred_element_type=jnp.float32) + bf2[...],
                     0.0)
    out[...] = jax.nn.sigmoid(
        jnp.dot(h2, Wf3[...], preferred_element_type=jnp.float32) + bf3[...])


def _tc_final(sumP, maxP, batch2d, xt, Wt1, bt1, Wt2, bt2,
              Wf1, bf1, Wf2, bf2, Wf3, bf3):
    return pl.pallas_call(
        _final_body,
        out_shape=jax.ShapeDtypeStruct((G, 1), jnp.float32),
    )(sumP, maxP, batch2d, xt, Wt1, bt1, Wt2, bt2,
      Wf1, bf1, Wf2, bf2, Wf3, bf3)


def kernel(x_graph, edge_index, batch, x_tabular,
           W_g1, b_g1, W_g2, b_g2, W_g3, b_g3, W_g4, b_g4,
           W_t1, b_t1, W_t2, b_t2,
           W_f1, b_f1, W_f2, b_f2, W_f3, b_f3):
    f32 = jnp.float32
    xgpad = jnp.pad(x_graph.astype(f32), ((0, NPAD - N), (0, 0)))
    padlen = EPAD - E
    pad_ix = jnp.full((padlen,), N, jnp.int32)
    srcp = jnp.concatenate([edge_index[0].astype(jnp.int32), pad_ix]
                           ).reshape(32, NCHUNK, CHUNK)
    dstp = jnp.concatenate([edge_index[1].astype(jnp.int32), pad_ix]
                           ).reshape(32, NCHUNK, CHUNK)
    batchp = jnp.concatenate(
        [batch.astype(jnp.int32), jnp.full((NPAD - N,), G, jnp.int32)])
    zeros1d = jnp.zeros((NPAD,), f32)
    zrows = jnp.zeros((RPT, F), f32)

    indeg = _sc_indeg(dstp, zeros1d)
    indegT = indeg.T

    hp1, dcol = _tc_mm1(indegT, xgpad, W_g1)
    brows = [b.reshape(1, F) for b in (b_g1, b_g2, b_g3)]

    S = _sc_msg(hp1, srcp, dstp, zrows)
    hp2 = _tc_mid(S[0], S[1], hp1, dcol, brows[0], W_g2)
    S = _sc_msg(hp2, srcp, dstp, zrows)
    hp3 = _tc_mid(S[0], S[1], hp2, dcol, brows[1], W_g3)
    S = _sc_msg(hp3, srcp, dstp, zrows)
    hp4 = _tc_mid(S[0], S[1], hp3, dcol, brows[2], W_g4)
    S = _sc_msg(hp4, srcp, dstp, zrows)

    sumP, maxP = _sc_pool(S[0], S[1], hp4, dcol.reshape(NPAD), b_g4,
                          batchp, zrows)
    sumP = sumP.reshape(32, GP, F)
    maxP = maxP.reshape(32, GP, F)
    batch2d = batchp.reshape(NPAD // 128, 128)

    return _tc_final(sumP, maxP, batch2d, x_tabular,
                     W_t1, b_t1.reshape(1, -1), W_t2, b_t2.reshape(1, -1),
                     W_f1, b_f1.reshape(1, -1), W_f2, b_f2.reshape(1, -1),
                     W_f3, b_f3.reshape(1, -1))
